# baseline (speedup 1.0000x reference)
import functools

import jax
import jax.numpy as jnp
from jax import lax
from jax.experimental import pallas as pl
from jax.experimental.pallas import tpu as pltpu
from jax.experimental.pallas import tpu_sc as plsc


_N, _S, _D = 2048, 4, 4096
_NW = 32
_RPW = _N // _NW
_L = 16


def _sc_body(x_hbm, t_hbm, o_hbm, xbuf, tbuf):
    c = lax.axis_index("c")
    s = lax.axis_index("s")
    wid = s * 2 + c
    base = wid * _RPW

    def row(i, carry):
        r = base + i
        pltpu.sync_copy(x_hbm.at[r], xbuf)
        pltpu.sync_copy(t_hbm.at[r], tbuf)

        def chunk(j, _):
            off = j * _L
            t = tbuf[pl.ds(off, _L)]
            for rr in range(_S):
                xbuf[rr, pl.ds(off, _L)] = xbuf[rr, pl.ds(off, _L)] + t
            return 0

        lax.fori_loop(0, _D // _L, chunk, 0)
        pltpu.sync_copy(xbuf, o_hbm.at[r])
        return 0

    lax.fori_loop(0, _RPW, row, 0)


@jax.jit
def kernel(x, table):
    mesh = plsc.VectorSubcoreMesh(core_axis_name="c", subcore_axis_name="s")
    sc_fn = pl.kernel(
        _sc_body,
        mesh=mesh,
        out_type=jax.ShapeDtypeStruct((_N, _S, _D), jnp.float32),
        scratch_types=[
            pltpu.VMEM((_S, _D), jnp.float32),
            pltpu.VMEM((_D,), jnp.float32),
        ],
    )
    return sc_fn(x, table)


# speedup vs baseline: 1.1646x; 1.1646x over previous
import functools

import jax
import jax.numpy as jnp
from jax import lax
from jax.experimental import pallas as pl
from jax.experimental.pallas import tpu as pltpu
from jax.experimental.pallas import tpu_sc as plsc


_N, _S, _D = 2048, 4, 4096
_NW = 32
_RPW = _N // _NW
_L = 16


def _add_row(xb, tb):
    def chunk(j, _):
        off = pl.multiple_of(j * _L, _L)
        t = tb[pl.ds(off, _L)]
        for rr in range(_S):
            xb[rr, pl.ds(off, _L)] = xb[rr, pl.ds(off, _L)] + t
        return 0

    lax.fori_loop(0, _D // _L, chunk, 0, unroll=2)


def _sc_body(x_hbm, t_hbm, o_hbm, xb0, xb1, tb0, tb1,
             isem0, isem1, osem0, osem1):
    c = lax.axis_index("c")
    s = lax.axis_index("s")
    wid = s * 2 + c
    base = wid * _RPW
    xbufs = (xb0, xb1)
    tbufs = (tb0, tb1)
    isems = (isem0, isem1)
    osems = (osem0, osem1)

    def start_in(slot, r):
        pltpu.make_async_copy(x_hbm.at[r], xbufs[slot], isems[slot]).start()
        pltpu.make_async_copy(t_hbm.at[r], tbufs[slot], isems[slot]).start()

    def wait_in(slot, r):
        pltpu.make_async_copy(x_hbm.at[r], xbufs[slot], isems[slot]).wait()
        pltpu.make_async_copy(t_hbm.at[r], tbufs[slot], isems[slot]).wait()

    def start_out(slot, r):
        pltpu.make_async_copy(xbufs[slot], o_hbm.at[r], osems[slot]).start()

    def wait_out(slot, r):
        pltpu.make_async_copy(xbufs[slot], o_hbm.at[r], osems[slot]).wait()

    start_in(0, base)
    start_in(1, base + 1)

    def pair(k, _):
        r0 = base + 2 * k
        for slot in range(2):
            r = r0 + slot
            wait_in(slot, r)
            _add_row(xbufs[slot], tbufs[slot])
            start_out(slot, r)

        @pl.when(k < _RPW // 2 - 1)
        def _refill():
            for slot in range(2):
                r = r0 + slot
                wait_out(slot, r)
                start_in(slot, r + 2)

        return 0

    lax.fori_loop(0, _RPW // 2, pair, 0)
    wait_out(0, base + _RPW - 2)
    wait_out(1, base + _RPW - 1)


@jax.jit
def kernel(x, table):
    mesh = plsc.VectorSubcoreMesh(core_axis_name="c", subcore_axis_name="s")
    sc_fn = pl.kernel(
        _sc_body,
        mesh=mesh,
        out_type=jax.ShapeDtypeStruct((_N, _S, _D), jnp.float32),
        scratch_types=[
            pltpu.VMEM((_S, _D), jnp.float32),
            pltpu.VMEM((_S, _D), jnp.float32),
            pltpu.VMEM((_D,), jnp.float32),
            pltpu.VMEM((_D,), jnp.float32),
            pltpu.SemaphoreType.DMA,
            pltpu.SemaphoreType.DMA,
            pltpu.SemaphoreType.DMA,
            pltpu.SemaphoreType.DMA,
        ],
    )
    return sc_fn(x, table)
